# baseline (speedup 1.0000x reference)
import functools

import jax
import jax.numpy as jnp
from jax import lax
from jax.experimental import pallas as pl
from jax.experimental.pallas import tpu as pltpu
from jax.experimental.pallas import tpu_sc as plsc

_S = 1024
_D = 128
_T = 2 * _S - 1


def kernel(x, table):
    del x

    mesh = plsc.VectorSubcoreMesh(core_axis_name="c", subcore_axis_name="s")

    @functools.partial(
        pl.kernel,
        out_type=jax.ShapeDtypeStruct((_S, _S, _D), jnp.float32),
        mesh=mesh,
        scratch_types=[pltpu.MemorySpace.VMEM_SHARED((_T, _D), jnp.float32)],
    )
    def run(table_hbm, out_hbm, shared):
        cid = lax.axis_index("c")
        sid = lax.axis_index("s")

        @pl.when(sid == 0)
        def _stage():
            pltpu.sync_copy(table_hbm, shared)

        plsc.subcore_barrier()

        wid = sid * 2 + cid
        rows_per_w = _S // 32
        base = wid * rows_per_w
        for r in range(rows_per_w):
            i = base + r
            start = (_S - 1) - i
            pltpu.sync_copy(shared.at[pl.ds(start, _S)], out_hbm.at[i])

    return run(table)
